# Initial kernel scaffold; baseline (speedup 1.0000x reference)
import jax
import jax.numpy as jnp
from jax.experimental import pallas as pl


def kernel(x, adjacency, edge_index, conv_W, conv_b, norm_g, norm_b, normff_g, normff_b, ff_W1, ff_b1, ff_W2, ff_b2):
    raise NotImplementedError("write your pallas kernel here")


# speedup vs baseline: 6.6532x; 6.6532x over previous
import dataclasses
import functools

import jax
import jax.numpy as jnp
from jax import lax
from jax.experimental import pallas as pl
from jax.experimental.pallas import tpu as pltpu
from jax.experimental.pallas import tpu_sc as plsc

_F32 = jnp.float32
_BF16 = jnp.bfloat16
_SQRT1_2 = 0.7071067811865476


def _sc_compiler_params():
    cp = pltpu.CompilerParams()
    if "needs_layout_passes" in pltpu.CompilerParams.__dataclass_fields__:
        cp = dataclasses.replace(cp, needs_layout_passes=False)
    return cp


def _gelu(v):
    return 0.5 * v * (1.0 + lax.erf(v * _SQRT1_2))


def _sc_degree(ei_cols, t):
    erows = ei_cols.shape[0]
    rows_per_w = erows // 16
    strip = t // 16
    mesh = plsc.VectorSubcoreMesh(core_axis_name="c", subcore_axis_name="s")

    @functools.partial(
        pl.kernel,
        out_type=jax.ShapeDtypeStruct((t,), _F32),
        mesh=mesh,
        scratch_types=[
            pltpu.VMEM((rows_per_w, 128), jnp.int32),
            pltpu.VMEM((rows_per_w, 128), jnp.int32),
            pltpu.VMEM((128,), _F32),
            pltpu.VMEM((t,), _F32),
            pltpu.VMEM((16, strip), _F32),
            pltpu.VMEM_SHARED((16 * t,), _F32),
        ],
        compiler_params=_sc_compiler_params(),
    )
    def k(ei_ref, deg_ref, cols_v, sidx_v, ones_v, tmp_v, part_v, shared):
        cid = lax.axis_index("c")
        w = lax.axis_index("s")

        @pl.loop(0, t, step=16)
        def _(i):
            tmp_v[pl.ds(i, 16)] = jnp.zeros((16,), _F32)

        @pl.loop(0, 128, step=16)
        def _(i):
            ones_v[pl.ds(i, 16)] = jnp.ones((16,), _F32)

        pltpu.sync_copy(tmp_v, shared.at[pl.ds(w * t, t)])

        pltpu.sync_copy(ei_ref.at[pl.ds(w * rows_per_w, rows_per_w), :], cols_v)

        @pl.loop(0, rows_per_w)
        def _(j):
            @pl.loop(0, 128, step=16)
            def _(kk):
                sidx_v[j, pl.ds(kk, 16)] = cols_v[j, pl.ds(kk, 16)] + w * t

        @pl.loop(0, rows_per_w)
        def _(j):
            pltpu.sync_copy(ones_v, shared.at[sidx_v.at[j]], add=True)

        plsc.subcore_barrier()

        @pl.when(cid == 0)
        def _():
            @pl.loop(0, 16)
            def _(j):
                pltpu.sync_copy(shared.at[pl.ds(j * t + w * strip, strip)],
                                part_v.at[j])

            @pl.loop(0, strip, step=16)
            def _(i):
                acc = jnp.zeros((16,), _F32)
                for j in range(16):
                    acc = acc + part_v[j, pl.ds(i, 16)]
                tmp_v[pl.ds(i, 16)] = acc

            pltpu.sync_copy(tmp_v.at[pl.ds(0, strip)],
                            deg_ref.at[pl.ds(w * strip, strip)])

    return k(ei_cols)


def _sc_build_a(ei_pack, dis, t):
    erows = ei_pack.shape[0]
    wrows = 16
    chunk = wrows * t
    npass = t // (32 * wrows)
    zlen = 4096
    grp = 16
    mesh = plsc.VectorSubcoreMesh(core_axis_name="c", subcore_axis_name="s")

    @functools.partial(
        pl.kernel,
        out_type=jax.ShapeDtypeStruct((t * t,), _F32),
        mesh=mesh,
        scratch_types=[
            pltpu.VMEM((erows, 128), jnp.int32),
            pltpu.VMEM((grp, 128), jnp.int32),
            pltpu.VMEM((grp, 128), _F32),
            pltpu.VMEM((t,), _F32),
            pltpu.VMEM((zlen,), _F32),
            pltpu.VMEM_SHARED((16 * chunk,), _F32),
        ],
        compiler_params=_sc_compiler_params(),
    )
    def k(pk_hbm, dis_hbm, a_ref, pk_v, sidx_v, sval_v, dis_v, zeros_v,
          shared):
        cid = lax.axis_index("c")
        sid = lax.axis_index("s")
        w = sid * 2 + cid
        rbase = sid * chunk

        @pl.loop(0, zlen, step=16)
        def _(i):
            zeros_v[pl.ds(i, 16)] = jnp.zeros((16,), _F32)

        pltpu.sync_copy(dis_hbm, dis_v)
        pltpu.sync_copy(pk_hbm, pk_v)

        @pl.loop(0, npass)
        def _(p):
            c0 = (p * 32 + w) * wrows

            for z in range(chunk // zlen):
                pltpu.sync_copy(zeros_v,
                                shared.at[pl.ds(rbase + z * zlen, zlen)])

            @pl.loop(0, erows, step=grp)
            def _(jg):
                @pl.loop(0, grp)
                def _(j):
                    @pl.loop(0, 128, step=16)
                    def _(kk):
                        pk = pk_v[jg + j, pl.ds(kk, 16)]
                        c = lax.shift_right_logical(pk, 12)
                        r = pk & 4095
                        inr = (c >= c0) & (c < c0 + wrows)
                        nrm = (plsc.load_gather(dis_v, [r]) *
                               plsc.load_gather(dis_v, [c]))
                        sidx_v[j, pl.ds(kk, 16)] = jnp.where(
                            inr, rbase + (c - c0) * t + r, rbase)
                        sval_v[j, pl.ds(kk, 16)] = jnp.where(inr, nrm, 0.0)

                @pl.loop(0, grp)
                def _(j):
                    pltpu.sync_copy(sval_v.at[j], shared.at[sidx_v.at[j]],
                                    add=True)

            plsc.subcore_barrier()
            pltpu.sync_copy(shared.at[pl.ds(rbase, chunk)],
                            a_ref.at[pl.ds(c0 * t, chunk)])

    return k(ei_pack, dis)


def _xw_body(x_ref, w_ref, o_ref):
    o_ref[...] = jnp.dot(x_ref[...].astype(_BF16), w_ref[...],
                         preferred_element_type=_F32).astype(_BF16)


def _tc_xw(x2f, w16):
    m = x2f.shape[0]
    blk = 8192
    return pl.pallas_call(
        _xw_body,
        grid=(m // blk,),
        in_specs=[pl.BlockSpec((blk, 128), lambda i: (i, 0)),
                  pl.BlockSpec((128, 128), lambda i: (0, 0))],
        out_specs=pl.BlockSpec((blk, 128), lambda i: (i, 0)),
        out_shape=jax.ShapeDtypeStruct((m, 128), _BF16),
    )(x2f, w16)


def _agg_body(a_ref, h_ref, x_ref, cb_ref, g_ref, b_ref, o_ref):
    acc = jnp.dot(a_ref[...], h_ref[...], preferred_element_type=_F32)
    cb = cb_ref[...]
    gm = g_ref[...]
    bt = b_ref[...]
    for g in range(acc.shape[1] // 128):
        sl = slice(g * 128, (g + 1) * 128)
        seg = _gelu(acc[:, sl] + cb)
        mu = jnp.mean(seg, axis=-1, keepdims=True)
        cd = seg - mu
        var = jnp.mean(cd * cd, axis=-1, keepdims=True)
        y = cd * lax.rsqrt(var + 1e-5)
        o_ref[:, sl] = x_ref[:, sl] + y * gm + bt


def _tc_agg(a16, h2, x2, cb, gm, bt):
    t, n2 = x2.shape
    cblk, bg = 512, 2
    bw = n2 // bg
    return pl.pallas_call(
        _agg_body,
        grid=(bg, t // cblk),
        in_specs=[
            pl.BlockSpec((cblk, t), lambda b, c: (c, 0)),
            pl.BlockSpec((t, bw), lambda b, c: (0, b)),
            pl.BlockSpec((cblk, bw), lambda b, c: (c, b)),
            pl.BlockSpec((1, 128), lambda b, c: (0, 0)),
            pl.BlockSpec((1, 128), lambda b, c: (0, 0)),
            pl.BlockSpec((1, 128), lambda b, c: (0, 0)),
        ],
        out_specs=pl.BlockSpec((cblk, bw), lambda b, c: (c, b)),
        out_shape=jax.ShapeDtypeStruct((t, n2), _F32),
    )(a16, h2, x2, cb, gm, bt)


def _ff_body(x_ref, g_ref, b_ref, w1_ref, b1_ref, w2_ref, b2_ref, o_ref):
    xo = x_ref[...]
    mu = jnp.mean(xo, axis=-1, keepdims=True)
    cd = xo - mu
    var = jnp.mean(cd * cd, axis=-1, keepdims=True)
    xn = cd * lax.rsqrt(var + 1e-5) * g_ref[...] + b_ref[...]
    t1 = jnp.dot(xn.astype(_BF16), w1_ref[...],
                 preferred_element_type=_F32) + b1_ref[...]
    t1 = _gelu(t1)
    t2 = jnp.dot(t1.astype(_BF16), w2_ref[...],
                 preferred_element_type=_F32) + b2_ref[...]
    o_ref[...] = xo + t2


def _tc_ff(xf, g, b, w1, b1, w2, b2):
    m, d = xf.shape
    dh = w1.shape[1]
    blk = 4096
    return pl.pallas_call(
        _ff_body,
        grid=(m // blk,),
        in_specs=[
            pl.BlockSpec((blk, d), lambda i: (i, 0)),
            pl.BlockSpec((1, d), lambda i: (0, 0)),
            pl.BlockSpec((1, d), lambda i: (0, 0)),
            pl.BlockSpec((d, dh), lambda i: (0, 0)),
            pl.BlockSpec((1, dh), lambda i: (0, 0)),
            pl.BlockSpec((dh, d), lambda i: (0, 0)),
            pl.BlockSpec((1, d), lambda i: (0, 0)),
        ],
        out_specs=pl.BlockSpec((blk, d), lambda i: (i, 0)),
        out_shape=jax.ShapeDtypeStruct((m, d), _F32),
    )(xf, g, b, w1, b1, w2, b2)


def kernel(x, adjacency, edge_index, conv_W, conv_b, norm_g, norm_b,
           normff_g, normff_b, ff_W1, ff_b1, ff_W2, ff_b2):
    del adjacency
    b, t, d = x.shape
    e = edge_index.shape[1]
    nl = conv_W.shape[0]

    ei_cols = edge_index[1].reshape(e // 128, 128)
    ei_pack = ((edge_index[1] << 12) | edge_index[0]).reshape(e // 128, 128)

    deg = _sc_degree(ei_cols, t)
    dis = jnp.where(deg > 0, lax.rsqrt(deg), 0.0)
    a16 = _sc_build_a(ei_pack, dis, t).reshape(t, t).astype(_BF16)

    x2 = x.transpose(1, 0, 2).reshape(t, b * d)
    for i in range(nl):
        h2 = _tc_xw(x2.reshape(t * b, d),
                    conv_W[i].astype(_BF16)).reshape(t, b * d)
        x2 = _tc_agg(a16, h2, x2, conv_b[i].reshape(1, d),
                     norm_g[i].reshape(1, d), norm_b[i].reshape(1, d))

    out = _tc_ff(x2.reshape(t * b, d), normff_g.reshape(1, d),
                 normff_b.reshape(1, d), ff_W1.astype(_BF16),
                 ff_b1.reshape(1, 4 * d), ff_W2.astype(_BF16),
                 ff_b2.reshape(1, d))
    return out.reshape(t, b, d).transpose(1, 0, 2)


# speedup vs baseline: 19.4709x; 2.9266x over previous
import dataclasses
import functools

import jax
import jax.numpy as jnp
from jax import lax
from jax.experimental import pallas as pl
from jax.experimental.pallas import tpu as pltpu
from jax.experimental.pallas import tpu_sc as plsc

_F32 = jnp.float32
_BF16 = jnp.bfloat16
_SQRT1_2 = 0.7071067811865476


def _sc_compiler_params():
    cp = pltpu.CompilerParams()
    if "needs_layout_passes" in pltpu.CompilerParams.__dataclass_fields__:
        cp = dataclasses.replace(cp, needs_layout_passes=False)
    return cp


def _gelu(v):
    return 0.5 * v * (1.0 + lax.erf(v * _SQRT1_2))


def _sc_degree(ei_cols, t):
    erows = ei_cols.shape[0]
    rows_per_w = erows // 16
    strip = t // 16
    mesh = plsc.VectorSubcoreMesh(core_axis_name="c", subcore_axis_name="s")

    @functools.partial(
        pl.kernel,
        out_type=jax.ShapeDtypeStruct((t,), _F32),
        mesh=mesh,
        scratch_types=[
            pltpu.VMEM((rows_per_w, 128), jnp.int32),
            pltpu.VMEM((rows_per_w, 128), jnp.int32),
            pltpu.VMEM((128,), _F32),
            pltpu.VMEM((t,), _F32),
            pltpu.VMEM((16, strip), _F32),
            pltpu.VMEM_SHARED((16 * t,), _F32),
        ],
        compiler_params=_sc_compiler_params(),
    )
    def k(ei_ref, deg_ref, cols_v, sidx_v, ones_v, tmp_v, part_v, shared):
        cid = lax.axis_index("c")
        w = lax.axis_index("s")

        @pl.loop(0, t, step=16)
        def _(i):
            tmp_v[pl.ds(i, 16)] = jnp.zeros((16,), _F32)

        @pl.loop(0, 128, step=16)
        def _(i):
            ones_v[pl.ds(i, 16)] = jnp.ones((16,), _F32)

        pltpu.sync_copy(tmp_v, shared.at[pl.ds(w * t, t)])

        pltpu.sync_copy(ei_ref.at[pl.ds(w * rows_per_w, rows_per_w), :], cols_v)

        @pl.loop(0, rows_per_w)
        def _(j):
            @pl.loop(0, 128, step=16)
            def _(kk):
                sidx_v[j, pl.ds(kk, 16)] = cols_v[j, pl.ds(kk, 16)] + w * t

        @pl.loop(0, rows_per_w)
        def _(j):
            pltpu.sync_copy(ones_v, shared.at[sidx_v.at[j]], add=True)

        plsc.subcore_barrier()

        @pl.when(cid == 0)
        def _():
            @pl.loop(0, 16)
            def _(j):
                pltpu.sync_copy(shared.at[pl.ds(j * t + w * strip, strip)],
                                part_v.at[j])

            @pl.loop(0, strip, step=16)
            def _(i):
                acc = jnp.zeros((16,), _F32)
                for j in range(16):
                    acc = acc + part_v[j, pl.ds(i, 16)]
                tmp_v[pl.ds(i, 16)] = acc

            pltpu.sync_copy(tmp_v.at[pl.ds(0, strip)],
                            deg_ref.at[pl.ds(w * strip, strip)])

    return k(ei_cols)


def _sc_build_a(ei_pack, dis, t):
    e = ei_pack.shape[0]
    wrows = 8
    wspan = t // 32
    npass = wspan // wrows
    chunk = wrows * t
    grp = 16
    cap = e + grp * 128
    sentinel = 0x7FFF0000
    mesh = plsc.VectorSubcoreMesh(core_axis_name="c", subcore_axis_name="s")

    @functools.partial(
        pl.kernel,
        out_type=jax.ShapeDtypeStruct((t * t,), _F32),
        mesh=mesh,
        scratch_types=[
            pltpu.VMEM((e,), jnp.int32),
            pltpu.VMEM((cap,), jnp.int32),
            pltpu.VMEM((grp, 128), jnp.int32),
            pltpu.VMEM((grp, 128), _F32),
            pltpu.VMEM((t,), _F32),
            pltpu.VMEM((t,), _F32),
            pltpu.VMEM_SHARED((16 * chunk,), _F32),
        ],
        compiler_params=_sc_compiler_params(),
    )
    def k(pk_hbm, dis_hbm, a_ref, pk_v, list_v, sidx_v, sval_v, dis_v,
          zeros_v, shared):
        cid = lax.axis_index("c")
        sid = lax.axis_index("s")
        w = sid * 2 + cid
        rbase = sid * chunk
        wbase = w * wspan

        @pl.loop(0, t, step=16)
        def _(i):
            zeros_v[pl.ds(i, 16)] = jnp.zeros((16,), _F32)

        @pl.loop(0, cap, step=16)
        def _(i):
            list_v[pl.ds(i, 16)] = jnp.full((16,), sentinel, jnp.int32)

        pltpu.sync_copy(dis_hbm, dis_v)
        pltpu.sync_copy(pk_hbm, pk_v)

        @pl.loop(0, e, step=16, init_carry=jnp.int32(0))
        def pos(i, pos):
            pk = pk_v[pl.ds(i, 16)]
            c = lax.shift_right_logical(pk, 12)
            m = (c >= wbase) & (c < wbase + wspan)
            mi = m.astype(jnp.int32)
            dst = jnp.where(m, pos + plsc.cumsum(mi) - 1, 0)
            plsc.store_scatter(list_v, [dst], jnp.where(m, pk, sentinel),
                               mask=m)
            return pos + jnp.sum(mi)

        ngroups = (pos + grp * 128 - 1) // (grp * 128)

        @pl.loop(0, npass)
        def _(p):
            c0 = wbase + p * wrows

            for z in range(chunk // t):
                pltpu.sync_copy(zeros_v,
                                shared.at[pl.ds(rbase + z * t, t)])

            @pl.loop(0, ngroups)
            def _(g):
                @pl.loop(0, grp)
                def _(j):
                    @pl.loop(0, 128, step=16)
                    def _(kk):
                        pk = list_v[pl.ds(g * (grp * 128) + j * 128 + kk, 16)]
                        c = lax.shift_right_logical(pk, 12)
                        r = pk & 4095
                        inr = (c >= c0) & (c < c0 + wrows)
                        nrm = (plsc.load_gather(dis_v, [r]) *
                               plsc.load_gather(dis_v, [c & (t - 1)]))
                        sidx_v[j, pl.ds(kk, 16)] = jnp.where(
                            inr, rbase + (c - c0) * t + r, rbase)
                        sval_v[j, pl.ds(kk, 16)] = jnp.where(inr, nrm, 0.0)

                @pl.loop(0, grp)
                def _(j):
                    pltpu.sync_copy(sval_v.at[j], shared.at[sidx_v.at[j]],
                                    add=True)

            plsc.subcore_barrier()
            pltpu.sync_copy(shared.at[pl.ds(rbase, chunk)],
                            a_ref.at[pl.ds(c0 * t, chunk)])

    return k(ei_pack, dis)


def _xw_body(x_ref, w_ref, o_ref):
    o_ref[...] = jnp.dot(x_ref[...].astype(_BF16), w_ref[...],
                         preferred_element_type=_F32).astype(_BF16)


def _tc_xw(x2f, w16):
    m = x2f.shape[0]
    blk = 8192
    return pl.pallas_call(
        _xw_body,
        grid=(m // blk,),
        in_specs=[pl.BlockSpec((blk, 128), lambda i: (i, 0)),
                  pl.BlockSpec((128, 128), lambda i: (0, 0))],
        out_specs=pl.BlockSpec((blk, 128), lambda i: (i, 0)),
        out_shape=jax.ShapeDtypeStruct((m, 128), _BF16),
    )(x2f, w16)


def _agg_body(a_ref, h_ref, x_ref, cb_ref, g_ref, b_ref, o_ref):
    acc = jnp.dot(a_ref[...], h_ref[...], preferred_element_type=_F32)
    cb = cb_ref[...]
    gm = g_ref[...]
    bt = b_ref[...]
    for g in range(acc.shape[1] // 128):
        sl = slice(g * 128, (g + 1) * 128)
        seg = _gelu(acc[:, sl] + cb)
        mu = jnp.mean(seg, axis=-1, keepdims=True)
        cd = seg - mu
        var = jnp.mean(cd * cd, axis=-1, keepdims=True)
        y = cd * lax.rsqrt(var + 1e-5)
        o_ref[:, sl] = x_ref[:, sl] + y * gm + bt


def _tc_agg(a16, h2, x2, cb, gm, bt):
    t, n2 = x2.shape
    cblk, bg = 512, 2
    bw = n2 // bg
    return pl.pallas_call(
        _agg_body,
        grid=(bg, t // cblk),
        in_specs=[
            pl.BlockSpec((cblk, t), lambda b, c: (c, 0)),
            pl.BlockSpec((t, bw), lambda b, c: (0, b)),
            pl.BlockSpec((cblk, bw), lambda b, c: (c, b)),
            pl.BlockSpec((1, 128), lambda b, c: (0, 0)),
            pl.BlockSpec((1, 128), lambda b, c: (0, 0)),
            pl.BlockSpec((1, 128), lambda b, c: (0, 0)),
        ],
        out_specs=pl.BlockSpec((cblk, bw), lambda b, c: (c, b)),
        out_shape=jax.ShapeDtypeStruct((t, n2), _F32),
    )(a16, h2, x2, cb, gm, bt)


def _ff_body(x_ref, g_ref, b_ref, w1_ref, b1_ref, w2_ref, b2_ref, o_ref):
    xo = x_ref[...]
    mu = jnp.mean(xo, axis=-1, keepdims=True)
    cd = xo - mu
    var = jnp.mean(cd * cd, axis=-1, keepdims=True)
    xn = cd * lax.rsqrt(var + 1e-5) * g_ref[...] + b_ref[...]
    t1 = jnp.dot(xn.astype(_BF16), w1_ref[...],
                 preferred_element_type=_F32) + b1_ref[...]
    t1 = _gelu(t1)
    t2 = jnp.dot(t1.astype(_BF16), w2_ref[...],
                 preferred_element_type=_F32) + b2_ref[...]
    o_ref[...] = xo + t2


def _tc_ff(xf, g, b, w1, b1, w2, b2):
    m, d = xf.shape
    dh = w1.shape[1]
    blk = 4096
    return pl.pallas_call(
        _ff_body,
        grid=(m // blk,),
        in_specs=[
            pl.BlockSpec((blk, d), lambda i: (i, 0)),
            pl.BlockSpec((1, d), lambda i: (0, 0)),
            pl.BlockSpec((1, d), lambda i: (0, 0)),
            pl.BlockSpec((d, dh), lambda i: (0, 0)),
            pl.BlockSpec((1, dh), lambda i: (0, 0)),
            pl.BlockSpec((dh, d), lambda i: (0, 0)),
            pl.BlockSpec((1, d), lambda i: (0, 0)),
        ],
        out_specs=pl.BlockSpec((blk, d), lambda i: (i, 0)),
        out_shape=jax.ShapeDtypeStruct((m, d), _F32),
    )(xf, g, b, w1, b1, w2, b2)


def kernel(x, adjacency, edge_index, conv_W, conv_b, norm_g, norm_b,
           normff_g, normff_b, ff_W1, ff_b1, ff_W2, ff_b2):
    del adjacency
    b, t, d = x.shape
    e = edge_index.shape[1]
    nl = conv_W.shape[0]

    ei_cols = edge_index[1].reshape(e // 128, 128)
    ei_pack = (edge_index[1] << 12) | edge_index[0]

    deg = _sc_degree(ei_cols, t)
    dis = jnp.where(deg > 0, lax.rsqrt(deg), 0.0)
    a16 = _sc_build_a(ei_pack, dis, t).reshape(t, t).astype(_BF16)

    x2 = x.transpose(1, 0, 2).reshape(t, b * d)
    for i in range(nl):
        h2 = _tc_xw(x2.reshape(t * b, d),
                    conv_W[i].astype(_BF16)).reshape(t, b * d)
        x2 = _tc_agg(a16, h2, x2, conv_b[i].reshape(1, d),
                     norm_g[i].reshape(1, d), norm_b[i].reshape(1, d))

    out = _tc_ff(x2.reshape(t * b, d), normff_g.reshape(1, d),
                 normff_b.reshape(1, d), ff_W1.astype(_BF16),
                 ff_b1.reshape(1, 4 * d), ff_W2.astype(_BF16),
                 ff_b2.reshape(1, d))
    return out.reshape(t, b, d).transpose(1, 0, 2)
